# Initial kernel scaffold; baseline (speedup 1.0000x reference)
import functools

import jax
import jax.numpy as jnp
from jax import lax
from jax.experimental import pallas as pl
from jax.experimental.pallas import tpu as pltpu
from jax.experimental.pallas import tpu_sc as plsc

_N = 10000
_E = 320000
_F_IN = 128
_HID = 64
_NCLS = 16
_NGRAPH = 64

_NP = 10240
_CH = 128
_NW = 32
_CHUNKS_PER_W = 79
_NCHP = _NW * _CHUNKS_PER_W
_EP = _NCHP * _CH
_DEGW = 16
_RB = 2000
_NRB = 5

_mesh = plsc.VectorSubcoreMesh(core_axis_name="c", subcore_axis_name="s")


@functools.partial(
    pl.kernel,
    out_type=jax.ShapeDtypeStruct((2, _NP, _DEGW), jnp.float32),
    mesh=_mesh,
    scratch_types=[
        pltpu.VMEM((_CH,), jnp.int32),
        pltpu.VMEM((_CH, _DEGW), jnp.float32),
    ],
)
def _sc_degree(dst_hbm, ones_hbm, zero_hbm, out_hbm, idx_d, buf):
    c = lax.axis_index("c")
    s = lax.axis_index("s")
    w = s * 2 + c

    def scoped(deg_sp):
        pltpu.sync_copy(zero_hbm, buf)
        for k in range(5):
            base = s * 640 + k * _CH
            pltpu.sync_copy(buf, deg_sp.at[pl.ds(base, _CH)])
        pltpu.sync_copy(ones_hbm, buf)
        plsc.subcore_barrier()

        def body(j, carry):
            q = w * _CHUNKS_PER_W + j
            pltpu.sync_copy(dst_hbm.at[q], idx_d)
            pltpu.sync_copy(buf, deg_sp.at[idx_d], add=True)
            return carry

        lax.fori_loop(0, _CHUNKS_PER_W, body, 0)
        plsc.subcore_barrier()
        for k in range(5):
            base = s * 640 + k * _CH
            pltpu.sync_copy(deg_sp.at[pl.ds(base, _CH)], buf)
            pltpu.sync_copy(buf, out_hbm.at[c, pl.ds(base, _CH)])

    pl.run_scoped(scoped, pltpu.VMEM_SHARED((_NP, _DEGW), jnp.float32))


@functools.partial(
    pl.kernel,
    out_type=jax.ShapeDtypeStruct((2, _NP, _HID), jnp.float32),
    mesh=_mesh,
    scratch_types=[
        pltpu.VMEM((_CH,), jnp.int32),
        pltpu.VMEM((_CH,), jnp.int32),
        pltpu.VMEM((_CH, _HID), jnp.float32),
        pltpu.SemaphoreType.DMA,
    ],
)
def _sc_scatter(g_hbm, src_hbm, dst_hbm, zero_hbm, out_hbm,
                idx_s, idx_d, rows, sem):
    c = lax.axis_index("c")
    s = lax.axis_index("s")
    w = s * 2 + c

    def scoped(acc_sp):
        pltpu.sync_copy(zero_hbm, rows)
        for k in range(5):
            base = s * 640 + k * _CH
            pltpu.sync_copy(rows, acc_sp.at[pl.ds(base, _CH)])
        plsc.subcore_barrier()

        def body(j, carry):
            q = w * _CHUNKS_PER_W + j
            pltpu.sync_copy(src_hbm.at[q], idx_s)
            pltpu.sync_copy(dst_hbm.at[q], idx_d)
            pltpu.async_copy(g_hbm.at[idx_s], rows, sem).wait()
            pltpu.sync_copy(rows, acc_sp.at[idx_d], add=True)
            return carry

        lax.fori_loop(0, _CHUNKS_PER_W, body, 0)
        plsc.subcore_barrier()
        for k in range(5):
            base = s * 640 + k * _CH
            pltpu.sync_copy(acc_sp.at[pl.ds(base, _CH)], rows)
            pltpu.sync_copy(rows, out_hbm.at[c, pl.ds(base, _CH)])

    pl.run_scoped(scoped, pltpu.VMEM_SHARED((_NP, _HID), jnp.float32))


def _dinv_block(d0, d1):
    deg = 1.0 + d0[0, :, 0:1] + d1[0, :, 0:1]
    return lax.rsqrt(deg)


def _tc_in_body(x_ref, w_ref, d0_ref, d1_ref, o_ref):
    dinv = _dinv_block(d0_ref, d1_ref)
    h = jnp.dot(x_ref[...], w_ref[...], preferred_element_type=jnp.float32)
    o_ref[...] = h * dinv


def _tc_in(x, W1, degp):
    return pl.pallas_call(
        _tc_in_body,
        grid=(_NRB,),
        in_specs=[
            pl.BlockSpec((_RB, _F_IN), lambda i: (i, 0)),
            pl.BlockSpec((_F_IN, _HID), lambda i: (0, 0)),
            pl.BlockSpec((1, _RB, _DEGW), lambda i: (0, i, 0)),
            pl.BlockSpec((1, _RB, _DEGW), lambda i: (1, i, 0)),
        ],
        out_specs=pl.BlockSpec((_RB, _HID), lambda i: (i, 0)),
        out_shape=jax.ShapeDtypeStruct((_NP, _HID), jnp.float32),
    )(x, W1, degp, degp)


def _tc_mid_body(p0_ref, p1_ref, g_ref, d0_ref, d1_ref, b_ref, w_ref, o_ref):
    dinv = _dinv_block(d0_ref, d1_ref)
    acc = p0_ref[0] + p1_ref[0] + g_ref[...]
    h = jnp.maximum(acc * dinv + b_ref[...], 0.0)
    o_ref[...] = jnp.dot(h, w_ref[...], preferred_element_type=jnp.float32) * dinv


def _tc_mid(p, g, degp, b, W):
    return pl.pallas_call(
        _tc_mid_body,
        grid=(_NRB,),
        in_specs=[
            pl.BlockSpec((1, _RB, _HID), lambda i: (0, i, 0)),
            pl.BlockSpec((1, _RB, _HID), lambda i: (1, i, 0)),
            pl.BlockSpec((_RB, _HID), lambda i: (i, 0)),
            pl.BlockSpec((1, _RB, _DEGW), lambda i: (0, i, 0)),
            pl.BlockSpec((1, _RB, _DEGW), lambda i: (1, i, 0)),
            pl.BlockSpec((1, _HID), lambda i: (0, 0)),
            pl.BlockSpec((_HID, _HID), lambda i: (0, 0)),
        ],
        out_specs=pl.BlockSpec((_RB, _HID), lambda i: (i, 0)),
        out_shape=jax.ShapeDtypeStruct((_NP, _HID), jnp.float32),
    )(p, p, g, degp, degp, b.reshape(1, _HID), W)


def _tc_final_body(p0_ref, p1_ref, g_ref, d0_ref, d1_ref, b_ref, bt_ref,
                   wc_ref, bc_ref, o_ref, sums, counts):
    i = pl.program_id(0)
    dinv = _dinv_block(d0_ref, d1_ref)
    acc = p0_ref[0] + p1_ref[0] + g_ref[...]
    h = jnp.maximum(acc * dinv + b_ref[...], 0.0)
    bt = bt_ref[0, 0, :]
    gids = lax.broadcasted_iota(jnp.int32, (_RB, _NGRAPH), 1)
    oh = (bt[:, None] == gids).astype(jnp.float32)
    ps = lax.dot_general(oh, h, (((0,), (0,)), ((), ())),
                         preferred_element_type=jnp.float32)
    pc = lax.dot_general(oh, jnp.ones((_RB, _HID), jnp.float32),
                         (((0,), (0,)), ((), ())),
                         preferred_element_type=jnp.float32)

    @pl.when(i == 0)
    def _():
        sums[...] = jnp.zeros_like(sums)
        counts[...] = jnp.zeros_like(counts)

    sums[...] += ps
    counts[...] += pc
    pooled = sums[...] / jnp.maximum(counts[...], 1.0)
    o_ref[...] = (jnp.dot(pooled, wc_ref[...],
                          preferred_element_type=jnp.float32) + bc_ref[...])


def _tc_final(p, g, degp, b, batch3, Wc, bc):
    return pl.pallas_call(
        _tc_final_body,
        grid=(_NRB,),
        in_specs=[
            pl.BlockSpec((1, _RB, _HID), lambda i: (0, i, 0)),
            pl.BlockSpec((1, _RB, _HID), lambda i: (1, i, 0)),
            pl.BlockSpec((_RB, _HID), lambda i: (i, 0)),
            pl.BlockSpec((1, _RB, _DEGW), lambda i: (0, i, 0)),
            pl.BlockSpec((1, _RB, _DEGW), lambda i: (1, i, 0)),
            pl.BlockSpec((1, _HID), lambda i: (0, 0)),
            pl.BlockSpec((1, 1, _RB), lambda i: (i, 0, 0)),
            pl.BlockSpec((_HID, _NCLS), lambda i: (0, 0)),
            pl.BlockSpec((1, _NCLS), lambda i: (0, 0)),
        ],
        out_specs=pl.BlockSpec((_NGRAPH, _NCLS), lambda i: (0, 0)),
        out_shape=jax.ShapeDtypeStruct((_NGRAPH, _NCLS), jnp.float32),
        scratch_shapes=[
            pltpu.VMEM((_NGRAPH, _HID), jnp.float32),
            pltpu.VMEM((_NGRAPH, _HID), jnp.float32),
        ],
    )(p, p, g, degp, degp, b.reshape(1, _HID), batch3, Wc,
      bc.reshape(1, _NCLS))


def kernel(x, edge_index, edge_attr, batch, W1, b1, W2, b2, W3, b3, Wc, bc):
    pad = jnp.full((_EP - _E,), _N, dtype=jnp.int32)
    src2 = jnp.concatenate([edge_index[0], pad]).reshape(_NCHP, _CH)
    dst2 = jnp.concatenate([edge_index[1], pad]).reshape(_NCHP, _CH)
    ones16 = jnp.ones((_CH, _DEGW), jnp.float32)
    zero16 = jnp.zeros((_CH, _DEGW), jnp.float32)
    zero64 = jnp.zeros((_CH, _HID), jnp.float32)
    batch3 = batch.reshape(_NRB, 1, _RB)

    degp = _sc_degree(dst2, ones16, zero16)
    g1 = _tc_in(x, W1, degp)
    p1 = _sc_scatter(g1, src2, dst2, zero64)
    g2 = _tc_mid(p1, g1, degp, b1, W2)
    p2 = _sc_scatter(g2, src2, dst2, zero64)
    g3 = _tc_mid(p2, g2, degp, b2, W3)
    p3 = _sc_scatter(g3, src2, dst2, zero64)
    return _tc_final(p3, g3, degp, b3, batch3, Wc, bc)


# speedup vs baseline: 12.8011x; 12.8011x over previous
import functools

import jax
import jax.numpy as jnp
from jax import lax
from jax.experimental import pallas as pl
from jax.experimental.pallas import tpu as pltpu
from jax.experimental.pallas import tpu_sc as plsc

_N = 10000
_E = 320000
_F_IN = 128
_HID = 64
_NCLS = 16
_NGRAPH = 64

_NP = 10240
_CH = 128
_NW = 32
_CHUNKS_PER_W = 79
_NCHP = _NW * _CHUNKS_PER_W
_EP = _NCHP * _CH
_DEGW = 16
_RB = 2000
_NRB = 5

_mesh = plsc.VectorSubcoreMesh(core_axis_name="c", subcore_axis_name="s")


@functools.partial(
    pl.kernel,
    out_type=jax.ShapeDtypeStruct((2, _NP, _DEGW), jnp.float32),
    mesh=_mesh,
    compiler_params=pltpu.CompilerParams(use_tc_tiling_on_sc=False),
    scratch_types=[
        pltpu.VMEM((_CH,), jnp.int32),
        pltpu.VMEM((_CH, _DEGW), jnp.float32),
        pltpu.VMEM_SHARED((_NP, _DEGW), jnp.float32),
    ],
)
def _sc_degree(dst_hbm, ones_hbm, zero_hbm, out_hbm, idx_d, buf, deg_sp):
    c = lax.axis_index("c")
    s = lax.axis_index("s")
    w = s * 2 + c

    pltpu.sync_copy(zero_hbm, buf)
    for k in range(5):
        base = s * 640 + k * _CH
        pltpu.sync_copy(buf, deg_sp.at[pl.ds(base, _CH)])
    pltpu.sync_copy(ones_hbm, buf)
    plsc.subcore_barrier()

    def body(j, carry):
        q = w * _CHUNKS_PER_W + j
        pltpu.sync_copy(dst_hbm.at[q], idx_d)
        pltpu.sync_copy(buf, deg_sp.at[idx_d], add=True)
        return carry

    lax.fori_loop(0, _CHUNKS_PER_W, body, 0)
    plsc.subcore_barrier()
    for k in range(5):
        base = s * 640 + k * _CH
        pltpu.sync_copy(deg_sp.at[pl.ds(base, _CH)], buf)
        pltpu.sync_copy(buf, out_hbm.at[c, pl.ds(base, _CH)])


@functools.partial(
    pl.kernel,
    out_type=jax.ShapeDtypeStruct((2, _NP, _HID), jnp.float32),
    mesh=_mesh,
    compiler_params=pltpu.CompilerParams(use_tc_tiling_on_sc=False),
    scratch_types=[
        pltpu.VMEM((_CH,), jnp.int32),
        pltpu.VMEM((_CH,), jnp.int32),
        pltpu.VMEM((_CH, _HID), jnp.float32),
        pltpu.VMEM_SHARED((_NP, _HID), jnp.float32),
        pltpu.SemaphoreType.DMA,
    ],
)
def _sc_scatter(g_hbm, src_hbm, dst_hbm, zero_hbm, out_hbm,
                idx_s, idx_d, rows, acc_sp, sem):
    c = lax.axis_index("c")
    s = lax.axis_index("s")
    w = s * 2 + c

    pltpu.sync_copy(zero_hbm, rows)
    for k in range(5):
        base = s * 640 + k * _CH
        pltpu.sync_copy(rows, acc_sp.at[pl.ds(base, _CH)])
    plsc.subcore_barrier()

    def body(j, carry):
        q = w * _CHUNKS_PER_W + j
        pltpu.sync_copy(src_hbm.at[q], idx_s)
        pltpu.sync_copy(dst_hbm.at[q], idx_d)
        pltpu.async_copy(g_hbm.at[idx_s], rows, sem).wait()
        pltpu.sync_copy(rows, acc_sp.at[idx_d], add=True)
        return carry

    lax.fori_loop(0, _CHUNKS_PER_W, body, 0)
    plsc.subcore_barrier()
    for k in range(5):
        base = s * 640 + k * _CH
        pltpu.sync_copy(acc_sp.at[pl.ds(base, _CH)], rows)
        pltpu.sync_copy(rows, out_hbm.at[c, pl.ds(base, _CH)])


def _dinv_block(d0, d1):
    deg = 1.0 + d0[0, :, 0:1] + d1[0, :, 0:1]
    return lax.rsqrt(deg)


def _tc_in_body(x_ref, w_ref, d0_ref, d1_ref, o_ref):
    dinv = _dinv_block(d0_ref, d1_ref)
    h = jnp.dot(x_ref[...], w_ref[...], preferred_element_type=jnp.float32)
    o_ref[...] = h * dinv


def _tc_in(x, W1, degp):
    return pl.pallas_call(
        _tc_in_body,
        grid=(_NRB,),
        in_specs=[
            pl.BlockSpec((_RB, _F_IN), lambda i: (i, 0)),
            pl.BlockSpec((_F_IN, _HID), lambda i: (0, 0)),
            pl.BlockSpec((1, _RB, _DEGW), lambda i: (0, i, 0)),
            pl.BlockSpec((1, _RB, _DEGW), lambda i: (1, i, 0)),
        ],
        out_specs=pl.BlockSpec((_RB, _HID), lambda i: (i, 0)),
        out_shape=jax.ShapeDtypeStruct((_NP, _HID), jnp.float32),
    )(x, W1, degp, degp)


def _tc_mid_body(p0_ref, p1_ref, g_ref, d0_ref, d1_ref, b_ref, w_ref, o_ref):
    dinv = _dinv_block(d0_ref, d1_ref)
    acc = p0_ref[0] + p1_ref[0] + g_ref[...]
    h = jnp.maximum(acc * dinv + b_ref[...], 0.0)
    o_ref[...] = jnp.dot(h, w_ref[...], preferred_element_type=jnp.float32) * dinv


def _tc_mid(p, g, degp, b, W):
    return pl.pallas_call(
        _tc_mid_body,
        grid=(_NRB,),
        in_specs=[
            pl.BlockSpec((1, _RB, _HID), lambda i: (0, i, 0)),
            pl.BlockSpec((1, _RB, _HID), lambda i: (1, i, 0)),
            pl.BlockSpec((_RB, _HID), lambda i: (i, 0)),
            pl.BlockSpec((1, _RB, _DEGW), lambda i: (0, i, 0)),
            pl.BlockSpec((1, _RB, _DEGW), lambda i: (1, i, 0)),
            pl.BlockSpec((1, _HID), lambda i: (0, 0)),
            pl.BlockSpec((_HID, _HID), lambda i: (0, 0)),
        ],
        out_specs=pl.BlockSpec((_RB, _HID), lambda i: (i, 0)),
        out_shape=jax.ShapeDtypeStruct((_NP, _HID), jnp.float32),
    )(p, p, g, degp, degp, b.reshape(1, _HID), W)


def _tc_final_body(p0_ref, p1_ref, g_ref, d0_ref, d1_ref, b_ref, bt_ref,
                   wc_ref, bc_ref, o_ref, sums, counts):
    i = pl.program_id(0)
    dinv = _dinv_block(d0_ref, d1_ref)
    acc = p0_ref[0] + p1_ref[0] + g_ref[...]
    h = jnp.maximum(acc * dinv + b_ref[...], 0.0)
    bt = bt_ref[0, 0, :]
    gids = lax.broadcasted_iota(jnp.int32, (_RB, _NGRAPH), 1)
    oh = (bt[:, None] == gids).astype(jnp.float32)
    ps = lax.dot_general(oh, h, (((0,), (0,)), ((), ())),
                         preferred_element_type=jnp.float32)
    pc = lax.dot_general(oh, jnp.ones((_RB, _HID), jnp.float32),
                         (((0,), (0,)), ((), ())),
                         preferred_element_type=jnp.float32)

    @pl.when(i == 0)
    def _():
        sums[...] = jnp.zeros_like(sums)
        counts[...] = jnp.zeros_like(counts)

    sums[...] += ps
    counts[...] += pc
    pooled = sums[...] / jnp.maximum(counts[...], 1.0)
    o_ref[...] = (jnp.dot(pooled, wc_ref[...],
                          preferred_element_type=jnp.float32) + bc_ref[...])


def _tc_final(p, g, degp, b, batch3, Wc, bc):
    return pl.pallas_call(
        _tc_final_body,
        grid=(_NRB,),
        in_specs=[
            pl.BlockSpec((1, _RB, _HID), lambda i: (0, i, 0)),
            pl.BlockSpec((1, _RB, _HID), lambda i: (1, i, 0)),
            pl.BlockSpec((_RB, _HID), lambda i: (i, 0)),
            pl.BlockSpec((1, _RB, _DEGW), lambda i: (0, i, 0)),
            pl.BlockSpec((1, _RB, _DEGW), lambda i: (1, i, 0)),
            pl.BlockSpec((1, _HID), lambda i: (0, 0)),
            pl.BlockSpec((1, 1, _RB), lambda i: (i, 0, 0)),
            pl.BlockSpec((_HID, _NCLS), lambda i: (0, 0)),
            pl.BlockSpec((1, _NCLS), lambda i: (0, 0)),
        ],
        out_specs=pl.BlockSpec((_NGRAPH, _NCLS), lambda i: (0, 0)),
        out_shape=jax.ShapeDtypeStruct((_NGRAPH, _NCLS), jnp.float32),
        scratch_shapes=[
            pltpu.VMEM((_NGRAPH, _HID), jnp.float32),
            pltpu.VMEM((_NGRAPH, _HID), jnp.float32),
        ],
    )(p, p, g, degp, degp, b.reshape(1, _HID), batch3, Wc,
      bc.reshape(1, _NCLS))


def kernel(x, edge_index, edge_attr, batch, W1, b1, W2, b2, W3, b3, Wc, bc):
    pad = jnp.full((_EP - _E,), _N, dtype=jnp.int32)
    src2 = jnp.concatenate([edge_index[0], pad]).reshape(_NCHP, _CH)
    dst2 = jnp.concatenate([edge_index[1], pad]).reshape(_NCHP, _CH)
    ones16 = jnp.ones((_CH, _DEGW), jnp.float32)
    zero16 = jnp.zeros((_CH, _DEGW), jnp.float32)
    zero64 = jnp.zeros((_CH, _HID), jnp.float32)
    batch3 = batch.reshape(_NRB, 1, _RB)

    degp = _sc_degree(dst2, ones16, zero16)
    g1 = _tc_in(x, W1, degp)
    p1 = _sc_scatter(g1, src2, dst2, zero64)
    g2 = _tc_mid(p1, g1, degp, b1, W2)
    p2 = _sc_scatter(g2, src2, dst2, zero64)
    g3 = _tc_mid(p2, g2, degp, b2, W3)
    p3 = _sc_scatter(g3, src2, dst2, zero64)
    return _tc_final(p3, g3, degp, b3, batch3, Wc, bc)
